# baseline (speedup 1.0000x reference)
import functools

import jax
import jax.numpy as jnp
from jax import lax
from jax.experimental import pallas as pl
from jax.experimental.pallas import tpu as pltpu
from jax.experimental.pallas import tpu_sc as plsc

B, T, D = 16, 4096, 1024
NC, NS = 2, 16
NW = NC * NS
HALF = T // 2
ZROWS = 64

_mesh = plsc.VectorSubcoreMesh(core_axis_name="c", subcore_axis_name="s")


@functools.partial(
    pl.kernel,
    out_type=(
        jax.ShapeDtypeStruct((B,), jnp.int32),
        jax.ShapeDtypeStruct((B,), jnp.int32),
        jax.ShapeDtypeStruct((B,), jnp.int32),
        jax.ShapeDtypeStruct((B,), jnp.int32),
    ),
    mesh=_mesh,
    scratch_types=[
        pltpu.VMEM((B * T,), jnp.int32),
        pltpu.VMEM((B,), jnp.int32),
        pltpu.VMEM((B,), jnp.int32),
        pltpu.VMEM((B,), jnp.int32),
        pltpu.VMEM((B,), jnp.int32),
        pltpu.VMEM((B,), jnp.int32),
    ],
    compiler_params=pltpu.CompilerParams(needs_layout_passes=False, use_tc_tiling_on_sc=False),
)
def _control(mask_hbm, sl_hbm, si_hbm, ri_hbm, nv_hbm,
             mask_v, rest_v, sl_v, si_v, ri_v, nv_v):
    wid = lax.axis_index("s") * NC + lax.axis_index("c")

    @pl.when(wid == 0)
    def _():
        pltpu.sync_copy(mask_hbm, mask_v)
        lane = lax.iota(jnp.int32, B)

        lo = jnp.zeros((B,), jnp.int32)
        step = T
        while step >= 1:
            cand = lo + step
            ok = cand <= T
            idx = lane * T + jnp.minimum(cand, T) - 1
            probe = plsc.load_gather(mask_v, [idx])
            lo = jnp.where(ok & (probe > 0), cand, lo)
            step //= 2
        lengths = lo

        keys = lengths * B + (B - 1 - lane)
        sk, sv = plsc.sort_key_val(keys, lane, descending=True)
        sorted_len = lax.shift_right_logical(sk, 4)

        rest_v[...] = lane
        plsc.store_scatter(rest_v, [sv], lane)

        nv = jnp.sum(jnp.where(lengths > 0, 1, 0).astype(jnp.int32))

        sl_v[...] = sorted_len
        si_v[...] = sv
        ri_v[...] = rest_v[...]
        nv_v[...] = jnp.full((B,), nv, jnp.int32)
        pltpu.sync_copy(sl_v, sl_hbm)
        pltpu.sync_copy(si_v, si_hbm)
        pltpu.sync_copy(ri_v, ri_hbm)
        pltpu.sync_copy(nv_v, nv_hbm)


@functools.partial(
    pl.kernel,
    out_type=jax.ShapeDtypeStruct((B, T, D), jnp.float32),
    mesh=_mesh,
    scratch_types=[
        pltpu.VMEM((B,), jnp.int32),
        pltpu.VMEM((B,), jnp.int32),
        pltpu.VMEM((ZROWS, D), jnp.float32),
    ],
    compiler_params=pltpu.CompilerParams(needs_layout_passes=False, use_tc_tiling_on_sc=False),
)
def _payload(in_hbm, si_hbm, sl_hbm, out_hbm, si_v, sl_v, zero_v):
    wid = lax.axis_index("s") * NC + lax.axis_index("c")
    r = wid // 2
    t0 = (wid % 2) * HALF

    pltpu.sync_copy(si_hbm, si_v)
    pltpu.sync_copy(sl_hbm, sl_v)
    lane = lax.iota(jnp.int32, B)
    sel = lane == r
    perm = jnp.sum(jnp.where(sel, si_v[...], 0))
    ln = jnp.sum(jnp.where(sel, sl_v[...], 0))
    valid = jnp.clip(ln - t0, 0, HALF)

    def _zrow(i, carry):
        def _zchunk(j, c2):
            zero_v[i, pl.ds(j * 16, 16)] = jnp.zeros((16,), jnp.float32)
            return c2
        return lax.fori_loop(0, D // 16, _zchunk, carry)
    lax.fori_loop(0, ZROWS, _zrow, 0)

    off = t0 + jnp.int32(0)
    for k in range(11, -1, -1):
        sz = 1 << k
        bit = (valid & sz) != 0

        @pl.when(bit)
        def _(off=off, sz=sz):
            pltpu.sync_copy(in_hbm.at[perm, pl.ds(off, sz)],
                            out_hbm.at[r, pl.ds(off, sz)])

        off = off + jnp.where(bit, sz, 0).astype(jnp.int32)

    invalid = HALF - valid
    nfull = invalid // ZROWS

    def _fill(i, o):
        pltpu.sync_copy(zero_v, out_hbm.at[r, pl.ds(o, ZROWS)])
        return o + ZROWS

    off = lax.fori_loop(0, nfull, _fill, off)
    rem = invalid - nfull * ZROWS
    for k in range(5, -1, -1):
        sz = 1 << k
        bit = (rem & sz) != 0

        @pl.when(bit)
        def _(off=off, sz=sz):
            pltpu.sync_copy(zero_v.at[pl.ds(0, sz)],
                            out_hbm.at[r, pl.ds(off, sz)])

        off = off + jnp.where(bit, sz, 0).astype(jnp.int32)


def kernel(inputs, mask):
    sl, si, ri, nv = _control(mask.reshape(B * T))
    packed = _payload(inputs, si, sl)
    return packed, sl, ri, si, nv[0]


# speedup vs baseline: 5.1658x; 5.1658x over previous
import functools

import jax
import jax.numpy as jnp
from jax import lax
from jax.experimental import pallas as pl
from jax.experimental.pallas import tpu as pltpu
from jax.experimental.pallas import tpu_sc as plsc

B, T, D = 16, 4096, 1024
NC, NS = 2, 16
NW = NC * NS
HALF = T // 2
ZROWS = 64

_mesh = plsc.VectorSubcoreMesh(core_axis_name="c", subcore_axis_name="s")


@functools.partial(
    pl.kernel,
    out_type=(
        jax.ShapeDtypeStruct((B,), jnp.int32),
        jax.ShapeDtypeStruct((B,), jnp.int32),
        jax.ShapeDtypeStruct((B,), jnp.int32),
        jax.ShapeDtypeStruct((B,), jnp.int32),
    ),
    mesh=_mesh,
    scratch_types=[
        pltpu.VMEM((B * T,), jnp.int32),
        pltpu.VMEM((B,), jnp.int32),
        pltpu.VMEM((B,), jnp.int32),
        pltpu.VMEM((B,), jnp.int32),
        pltpu.VMEM((B,), jnp.int32),
        pltpu.VMEM((B,), jnp.int32),
    ],
    compiler_params=pltpu.CompilerParams(needs_layout_passes=False, use_tc_tiling_on_sc=False),
)
def _control(mask_hbm, sl_hbm, si_hbm, ri_hbm, nv_hbm,
             mask_v, rest_v, sl_v, si_v, ri_v, nv_v):
    wid = lax.axis_index("s") * NC + lax.axis_index("c")

    @pl.when(wid == 0)
    def _():
        pltpu.sync_copy(mask_hbm, mask_v)
        lane = lax.iota(jnp.int32, B)

        lo = jnp.zeros((B,), jnp.int32)
        step = T
        while step >= 1:
            cand = lo + step
            ok = cand <= T
            idx = lane * T + jnp.minimum(cand, T) - 1
            probe = plsc.load_gather(mask_v, [idx])
            lo = jnp.where(ok & (probe > 0), cand, lo)
            step //= 2
        lengths = lo

        keys = lengths * B + (B - 1 - lane)
        sk, sv = plsc.sort_key_val(keys, lane, descending=True)
        sorted_len = lax.shift_right_logical(sk, 4)

        rest_v[...] = lane
        plsc.store_scatter(rest_v, [sv], lane)

        nv = jnp.sum(jnp.where(lengths > 0, 1, 0).astype(jnp.int32))

        sl_v[...] = sorted_len
        si_v[...] = sv
        ri_v[...] = rest_v[...]
        nv_v[...] = jnp.full((B,), nv, jnp.int32)
        pltpu.sync_copy(sl_v, sl_hbm)
        pltpu.sync_copy(si_v, si_hbm)
        pltpu.sync_copy(ri_v, ri_hbm)
        pltpu.sync_copy(nv_v, nv_hbm)


CH = 32
NCH = HALF // CH


@functools.partial(
    pl.kernel,
    out_type=jax.ShapeDtypeStruct((B, T, D), jnp.float32),
    mesh=_mesh,
    scratch_types=[
        pltpu.VMEM((B,), jnp.int32),
        pltpu.VMEM((B,), jnp.int32),
        pltpu.VMEM((CH, D), jnp.float32),
        pltpu.VMEM((CH, D), jnp.float32),
    ],
    compiler_params=pltpu.CompilerParams(needs_layout_passes=False, use_tc_tiling_on_sc=False),
)
def _payload(in_hbm, si_hbm, sl_hbm, out_hbm, si_v, sl_v, buf_v, zero_v):
    wid = lax.axis_index("s") * NC + lax.axis_index("c")
    r = wid // 2
    t0 = (wid % 2) * HALF

    pltpu.sync_copy(si_hbm, si_v)
    pltpu.sync_copy(sl_hbm, sl_v)
    lane = lax.iota(jnp.int32, B)
    sel = lane == r
    perm = jnp.sum(jnp.where(sel, si_v[...], 0))
    ln = jnp.sum(jnp.where(sel, sl_v[...], 0))
    valid = jnp.clip(ln - t0, 0, HALF)

    def _zrow(i, carry):
        def _zchunk(j, c2):
            zero_v[i, pl.ds(j * 16, 16)] = jnp.zeros((16,), jnp.float32)
            return c2
        return lax.fori_loop(0, D // 16, _zchunk, carry)
    lax.fori_loop(0, CH, _zrow, 0)

    def _chunk(c, carry):
        lo = c * CH
        off = t0 + lo
        need_read = lo < valid
        bnd = valid - lo

        @pl.when(need_read)
        def _():
            pltpu.sync_copy(in_hbm.at[perm, pl.ds(off, CH)], buf_v)

        @pl.when(need_read & (bnd < CH))
        def _():
            for k in range(CH):
                @pl.when(k >= bnd)
                def _(k=k):
                    def _z(j, c2):
                        buf_v[k, pl.ds(j * 16, 16)] = jnp.zeros(
                            (16,), jnp.float32)
                        return c2
                    lax.fori_loop(0, D // 16, _z, 0)

        @pl.when(need_read)
        def _():
            pltpu.sync_copy(buf_v, out_hbm.at[r, pl.ds(off, CH)])

        @pl.when(jnp.logical_not(need_read))
        def _():
            pltpu.sync_copy(zero_v, out_hbm.at[r, pl.ds(off, CH)])

        return carry

    lax.fori_loop(0, NCH, _chunk, 0)


def kernel(inputs, mask):
    sl, si, ri, nv = _control(mask.reshape(B * T))
    packed = _payload(inputs, si, sl)
    return packed, sl, ri, si, nv[0]


# speedup vs baseline: 18.6728x; 3.6147x over previous
import functools

import jax
import jax.numpy as jnp
from jax import lax
from jax.experimental import pallas as pl
from jax.experimental.pallas import tpu as pltpu
from jax.experimental.pallas import tpu_sc as plsc

B, T, D = 16, 4096, 1024
NC, NS = 2, 16
NW = NC * NS
HALF = T // 2
ZROWS = 64

_mesh = plsc.VectorSubcoreMesh(core_axis_name="c", subcore_axis_name="s")


@functools.partial(
    pl.kernel,
    out_type=(
        jax.ShapeDtypeStruct((B,), jnp.int32),
        jax.ShapeDtypeStruct((B,), jnp.int32),
        jax.ShapeDtypeStruct((B,), jnp.int32),
        jax.ShapeDtypeStruct((B,), jnp.int32),
    ),
    mesh=_mesh,
    scratch_types=[
        pltpu.VMEM((B * T,), jnp.int32),
        pltpu.VMEM((B,), jnp.int32),
        pltpu.VMEM((B,), jnp.int32),
        pltpu.VMEM((B,), jnp.int32),
        pltpu.VMEM((B,), jnp.int32),
        pltpu.VMEM((B,), jnp.int32),
    ],
    compiler_params=pltpu.CompilerParams(needs_layout_passes=False, use_tc_tiling_on_sc=False),
)
def _control(mask_hbm, sl_hbm, si_hbm, ri_hbm, nv_hbm,
             mask_v, rest_v, sl_v, si_v, ri_v, nv_v):
    wid = lax.axis_index("s") * NC + lax.axis_index("c")

    @pl.when(wid == 0)
    def _():
        pltpu.sync_copy(mask_hbm, mask_v)
        lane = lax.iota(jnp.int32, B)

        lo = jnp.zeros((B,), jnp.int32)
        step = T
        while step >= 1:
            cand = lo + step
            ok = cand <= T
            idx = lane * T + jnp.minimum(cand, T) - 1
            probe = plsc.load_gather(mask_v, [idx])
            lo = jnp.where(ok & (probe > 0), cand, lo)
            step //= 2
        lengths = lo

        keys = lengths * B + (B - 1 - lane)
        sk, sv = plsc.sort_key_val(keys, lane, descending=True)
        sorted_len = lax.shift_right_logical(sk, 4)

        rest_v[...] = lane
        plsc.store_scatter(rest_v, [sv], lane)

        nv = jnp.sum(jnp.where(lengths > 0, 1, 0).astype(jnp.int32))

        sl_v[...] = sorted_len
        si_v[...] = sv
        ri_v[...] = rest_v[...]
        nv_v[...] = jnp.full((B,), nv, jnp.int32)
        pltpu.sync_copy(sl_v, sl_hbm)
        pltpu.sync_copy(si_v, si_hbm)
        pltpu.sync_copy(ri_v, ri_hbm)
        pltpu.sync_copy(nv_v, nv_hbm)


CH = 32
NCH = HALF // CH


@functools.partial(
    pl.kernel,
    out_type=jax.ShapeDtypeStruct((B, T, D), jnp.float32),
    mesh=_mesh,
    scratch_types=[
        pltpu.VMEM((B,), jnp.int32),
        pltpu.VMEM((B,), jnp.int32),
        pltpu.VMEM((CH, D), jnp.float32),
        pltpu.VMEM((CH, D), jnp.float32),
    ],
    compiler_params=pltpu.CompilerParams(needs_layout_passes=False),
)
def _payload(in_hbm, si_hbm, sl_hbm, out_hbm, si_v, sl_v, buf_v, zero_v):
    wid = lax.axis_index("s") * NC + lax.axis_index("c")

    pltpu.sync_copy(si_hbm, si_v)
    pltpu.sync_copy(sl_hbm, sl_v)
    lane = lax.iota(jnp.int32, B)

    def _zrow(i, carry):
        def _zchunk(j, c2):
            zero_v[i, pl.ds(j * 16, 16)] = jnp.zeros((16,), jnp.float32)
            return c2
        return lax.fori_loop(0, D // 16, _zchunk, carry)
    lax.fori_loop(0, CH, _zrow, 0)

    cpr = T // CH

    def _chunk(k, carry):
        g = wid + NW * k
        r = g // cpr
        off = (g % cpr) * CH
        sel = lane == r
        perm = jnp.sum(jnp.where(sel, si_v[...], 0))
        ln = jnp.sum(jnp.where(sel, sl_v[...], 0))
        bnd = jnp.clip(ln - off, 0, CH)
        need_read = bnd > 0

        @pl.when(need_read)
        def _():
            pltpu.sync_copy(in_hbm.at[perm, pl.ds(off, CH)], buf_v)

        @pl.when(need_read & (bnd < CH))
        def _():
            for k2 in range(CH):
                @pl.when(k2 >= bnd)
                def _(k2=k2):
                    def _z(j, c2):
                        buf_v[k2, pl.ds(j * 16, 16)] = jnp.zeros(
                            (16,), jnp.float32)
                        return c2
                    lax.fori_loop(0, D // 16, _z, 0)

        @pl.when(need_read)
        def _():
            pltpu.sync_copy(buf_v, out_hbm.at[r, pl.ds(off, CH)])

        @pl.when(jnp.logical_not(need_read))
        def _():
            pltpu.sync_copy(zero_v, out_hbm.at[r, pl.ds(off, CH)])

        return carry

    lax.fori_loop(0, (B * cpr) // NW, _chunk, 0)


def kernel(inputs, mask):
    sl, si, ri, nv = _control(mask.reshape(B * T))
    packed = _payload(inputs, si, sl)
    return packed, sl, ri, si, nv[0]


# speedup vs baseline: 20.4669x; 1.0961x over previous
import functools

import jax
import jax.numpy as jnp
from jax import lax
from jax.experimental import pallas as pl
from jax.experimental.pallas import tpu as pltpu
from jax.experimental.pallas import tpu_sc as plsc

B, T, D = 16, 4096, 1024
NC, NS = 2, 16
NW = NC * NS
HALF = T // 2
ZROWS = 64

_mesh = plsc.VectorSubcoreMesh(core_axis_name="c", subcore_axis_name="s")


@functools.partial(
    pl.kernel,
    out_type=(
        jax.ShapeDtypeStruct((B,), jnp.int32),
        jax.ShapeDtypeStruct((B,), jnp.int32),
        jax.ShapeDtypeStruct((B,), jnp.int32),
        jax.ShapeDtypeStruct((B,), jnp.int32),
    ),
    mesh=_mesh,
    scratch_types=[
        pltpu.VMEM((B * T,), jnp.int32),
        pltpu.VMEM((B,), jnp.int32),
        pltpu.VMEM((B,), jnp.int32),
        pltpu.VMEM((B,), jnp.int32),
        pltpu.VMEM((B,), jnp.int32),
        pltpu.VMEM((B,), jnp.int32),
    ],
    compiler_params=pltpu.CompilerParams(needs_layout_passes=False, use_tc_tiling_on_sc=False),
)
def _control(mask_hbm, sl_hbm, si_hbm, ri_hbm, nv_hbm,
             mask_v, rest_v, sl_v, si_v, ri_v, nv_v):
    wid = lax.axis_index("s") * NC + lax.axis_index("c")

    @pl.when(wid == 0)
    def _():
        pltpu.sync_copy(mask_hbm, mask_v)
        lane = lax.iota(jnp.int32, B)

        lo = jnp.zeros((B,), jnp.int32)
        step = T
        while step >= 1:
            cand = lo + step
            ok = cand <= T
            idx = lane * T + jnp.minimum(cand, T) - 1
            probe = plsc.load_gather(mask_v, [idx])
            lo = jnp.where(ok & (probe > 0), cand, lo)
            step //= 2
        lengths = lo

        keys = lengths * B + (B - 1 - lane)
        sk, sv = plsc.sort_key_val(keys, lane, descending=True)
        sorted_len = lax.shift_right_logical(sk, 4)

        rest_v[...] = lane
        plsc.store_scatter(rest_v, [sv], lane)

        nv = jnp.sum(jnp.where(lengths > 0, 1, 0).astype(jnp.int32))

        sl_v[...] = sorted_len
        si_v[...] = sv
        ri_v[...] = rest_v[...]
        nv_v[...] = jnp.full((B,), nv, jnp.int32)
        pltpu.sync_copy(sl_v, sl_hbm)
        pltpu.sync_copy(si_v, si_hbm)
        pltpu.sync_copy(ri_v, ri_hbm)
        pltpu.sync_copy(nv_v, nv_hbm)


CH = 32
NCH = HALF // CH


@functools.partial(
    pl.kernel,
    out_type=jax.ShapeDtypeStruct((B, T, D), jnp.float32),
    mesh=_mesh,
    scratch_types=[
        pltpu.VMEM((B,), jnp.int32),
        pltpu.VMEM((B,), jnp.int32),
        pltpu.VMEM((CH, D), jnp.float32),
        pltpu.VMEM((CH, D), jnp.float32),
        pltpu.VMEM((CH, D), jnp.float32),
        pltpu.SemaphoreType.DMA,
        pltpu.SemaphoreType.DMA,
        pltpu.SemaphoreType.DMA,
        pltpu.SemaphoreType.DMA,
    ],
    compiler_params=pltpu.CompilerParams(needs_layout_passes=False),
)
def _payload(in_hbm, si_hbm, sl_hbm, out_hbm, si_v, sl_v,
             buf0_v, buf1_v, zero_v, gsem0, gsem1, ssem0, ssem1):
    wid = lax.axis_index("s") * NC + lax.axis_index("c")

    pltpu.sync_copy(si_hbm, si_v)
    pltpu.sync_copy(sl_hbm, sl_v)
    lane = lax.iota(jnp.int32, B)

    def _zrow(i, carry):
        def _zchunk(j, c2):
            zero_v[i, pl.ds(j * 16, 16)] = jnp.zeros((16,), jnp.float32)
            return c2
        return lax.fori_loop(0, D // 16, _zchunk, carry)
    lax.fori_loop(0, CH, _zrow, 0)

    cpr = T // CH
    nk = (B * cpr) // NW

    def _info(k):
        g = wid + NW * k
        r = g // cpr
        off = (g % cpr) * CH
        sel = lane == r
        perm = jnp.sum(jnp.where(sel, si_v[...], 0))
        ln = jnp.sum(jnp.where(sel, sl_v[...], 0))
        bnd = jnp.clip(ln - off, 0, CH)
        return r, off, perm, bnd

    def _half(k, buf, gsem, ssem, first):
        r, off, perm, bnd = _info(k)
        need_read = bnd > 0

        @pl.when(jnp.logical_not(first))
        def _():
            pltpu.make_async_copy(buf, out_hbm.at[r, pl.ds(off, CH)],
                                  ssem).wait()

        @pl.when(need_read)
        def _():
            pltpu.make_async_copy(in_hbm.at[perm, pl.ds(off, CH)], buf,
                                  gsem).start()
            pltpu.make_async_copy(in_hbm.at[perm, pl.ds(off, CH)], buf,
                                  gsem).wait()

        @pl.when(need_read & (bnd < CH))
        def _():
            for k2 in range(CH):
                @pl.when(k2 >= bnd)
                def _(k2=k2):
                    def _z(j, c2):
                        buf[k2, pl.ds(j * 16, 16)] = jnp.zeros(
                            (16,), jnp.float32)
                        return c2
                    lax.fori_loop(0, D // 16, _z, 0)

        @pl.when(need_read)
        def _():
            pltpu.make_async_copy(buf, out_hbm.at[r, pl.ds(off, CH)],
                                  ssem).start()

        @pl.when(jnp.logical_not(need_read))
        def _():
            pltpu.make_async_copy(zero_v, out_hbm.at[r, pl.ds(off, CH)],
                                  ssem).start()

    def _pair(i, carry):
        _half(2 * i, buf0_v, gsem0, ssem0, i == 0)
        _half(2 * i + 1, buf1_v, gsem1, ssem1, i == 0)
        return carry

    lax.fori_loop(0, nk // 2, _pair, 0)

    r, off, _, _ = _info(nk - 2)
    pltpu.make_async_copy(buf0_v, out_hbm.at[r, pl.ds(off, CH)], ssem0).wait()
    r, off, _, _ = _info(nk - 1)
    pltpu.make_async_copy(buf1_v, out_hbm.at[r, pl.ds(off, CH)], ssem1).wait()


def kernel(inputs, mask):
    sl, si, ri, nv = _control(mask.reshape(B * T))
    packed = _payload(inputs, si, sl)
    return packed, sl, ri, si, nv[0]
